# baseline (speedup 1.0000x reference)
import jax
import jax.numpy as jnp
from jax import lax
from jax.experimental import pallas as pl
from jax.experimental.pallas import tpu as pltpu

N = 512
N_IN = 128
N_OUT = 64
BI = 64
BJ = 64


def _fill_body(x_ref, w_ref, b_ref, o_ref, a_s, b_s):
    gi = pl.program_id(0)
    gj = pl.program_id(1)

    @pl.when((gi == 0) & (gj == 0))
    def _matmuls():
        x = x_ref[...]
        W1 = w_ref[:, :N_IN]
        W2 = w_ref[:, N_IN:]
        bh = 0.5 * b_ref[...]
        dn = (((1,), (1,)), ((), ()))
        a_s[...] = lax.dot_general(x, W1, dn, preferred_element_type=jnp.float32) + bh
        b_s[...] = lax.dot_general(x, W2, dn, preferred_element_type=jnp.float32) + bh

    ai = a_s[pl.ds(gi * BI, BI), :]
    bi = b_s[pl.ds(gi * BI, BI), :]
    aj = a_s[pl.ds(gj * BJ, BJ), :]
    bj = b_s[pl.ds(gj * BJ, BJ), :]

    @pl.when(gi < gj)
    def _upper():
        o_ref[...] = ai[:, None, :] + bj[None, :, :]

    @pl.when(gi > gj)
    def _lower():
        o_ref[...] = aj[None, :, :] + bi[:, None, :]

    @pl.when(gi == gj)
    def _diag():
        shape = (BI, BJ, N_OUT)
        R = lax.broadcasted_iota(jnp.int32, shape, 0)
        S = lax.broadcasted_iota(jnp.int32, shape, 1)
        out = jnp.where(S < R, aj[None, :, :] + bi[:, None, :],
                        ai[:, None, :] + bj[None, :, :])
        out = jnp.where(S == R, jnp.float32(0.0), out)
        o_ref[...] = out


def kernel(inputs, W, b):
    x = inputs
    b2 = b.reshape(1, N_OUT)
    out = pl.pallas_call(
        _fill_body,
        grid=(N // BI, N // BJ),
        in_specs=[
            pl.BlockSpec((N, N_IN), lambda i, j: (0, 0)),
            pl.BlockSpec((N_OUT, 2 * N_IN), lambda i, j: (0, 0)),
            pl.BlockSpec((1, N_OUT), lambda i, j: (0, 0)),
        ],
        out_specs=pl.BlockSpec((BI, BJ, N_OUT), lambda i, j: (i, j, 0)),
        out_shape=jax.ShapeDtypeStruct((N, N, N_OUT), jnp.float32),
        scratch_shapes=[
            pltpu.VMEM((N, N_OUT), jnp.float32),
            pltpu.VMEM((N, N_OUT), jnp.float32),
        ],
    )(x, W, b2)
    return out
